# baseline (speedup 1.0000x reference)
import jax
import jax.numpy as jnp
from jax import lax
from jax.experimental import pallas as pl
from jax.experimental.pallas import tpu as pltpu, tpu_sc as plsc

VOCAB = 100000
D = 128
SEQ = 200
BATCH = 1024
N = BATCH * SEQ
NC = 2
NS = 16
NW = NC * NS
ROWS_W = N // NW
CHUNK = 128
NCHUNK = ROWS_W // CHUNK
NBUF = 4
L = 16
NJ = D // L


def _body(tok_emb, tok_idx, seg_x, base_tab, delta, out,
          idx_v, sx_v, base_v, delta_v, rows_v, *sems):
    sid = lax.axis_index("s")
    wid = sid * NC + lax.axis_index("c")
    row0 = wid * ROWS_W

    gsems = sems[0:NBUF]
    wsems = sems[NBUF:2 * NBUF]
    psem = sems[2 * NBUF]

    pltpu.async_copy(tok_idx.at[wid], idx_v, psem)
    pltpu.async_copy(base_tab, base_v, psem)
    pltpu.async_copy(delta, delta_v, psem)
    pltpu.make_async_copy(tok_idx.at[wid], idx_v, psem).wait()
    pltpu.make_async_copy(base_tab, base_v, psem).wait()
    pltpu.make_async_copy(delta, delta_v, psem).wait()

    dregs = [delta_v[pl.ds(j * L, L)] for j in range(NJ)]

    def start_in(g, buf):
        pltpu.async_copy(tok_emb.at[idx_v.at[g]], rows_v.at[buf], gsems[buf])
        pltpu.async_copy(seg_x.at[wid].at[pl.ds(g * (CHUNK // 8), CHUNK // 8)],
                         sx_v.at[buf], gsems[buf])

    def wait_in(buf):
        pltpu.make_async_copy(
            tok_emb.at[idx_v.at[0]], rows_v.at[buf], gsems[buf]).wait()
        pltpu.make_async_copy(
            seg_x.at[0].at[pl.ds(0, CHUNK // 8)], sx_v.at[buf],
            gsems[buf]).wait()

    def start_write(g, buf):
        pltpu.async_copy(rows_v.at[buf],
                         out.at[pl.ds(row0 + g * CHUNK, CHUNK)], wsems[buf])

    def wait_write(buf):
        pltpu.make_async_copy(
            rows_v.at[buf], out.at[pl.ds(0, CHUNK)], wsems[buf]).wait()

    def compute(g, buf):
        rows = rows_v.at[buf]
        sx = sx_v.at[buf]
        pos0 = (g * CHUNK) % SEQ

        def row_fn(r, _):
            pos = lax.rem(pos0 + r, SEQ)
            pb = pos * D
            q, l = r // 8, r % 8
            s = sx[q, pl.ds(l * L, L)]
            for j in range(NJ):
                a = rows[r, pl.ds(j * L, L)]
                b = base_v[pl.ds(pb + j * L, L)]
                rows[r, pl.ds(j * L, L)] = a + b + s * dregs[j]
            return _

        lax.fori_loop(0, CHUNK, row_fn, None)

    for g in range(NCHUNK + 2):
        if g < NCHUNK:
            buf = g % NBUF
            if g >= NBUF:
                wait_write(buf)
            start_in(g, buf)
        if 0 <= g - 2 < NCHUNK:
            b = (g - 2) % NBUF
            wait_in(b)
            compute(g - 2, b)
            start_write(g - 2, b)
    for g in range(NCHUNK - NBUF, NCHUNK):
        wait_write(g % NBUF)


def kernel(x, segment_ids, tok_emb, seg_emb, pos_emb):
    tok_idx = x.astype(jnp.int32).reshape(NW, ROWS_W // 128, 128)
    seg_f = segment_ids.astype(jnp.float32).reshape(N, 1)
    seg_x = jnp.broadcast_to(seg_f, (N, L)).reshape(NW, ROWS_W // 8, 128)
    base_tab = (pos_emb[:SEQ] + seg_emb[0]).reshape(SEQ * D)
    delta = seg_emb[1] - seg_emb[0]

    mesh = plsc.VectorSubcoreMesh(core_axis_name="c", subcore_axis_name="s")
    out = pl.kernel(
        _body,
        out_type=jax.ShapeDtypeStruct((N, D), jnp.float32),
        mesh=mesh,
        scratch_types=[
            pltpu.VMEM((ROWS_W // 128, 128), jnp.int32),
            pltpu.VMEM((NBUF, CHUNK // 8, 128), jnp.float32),
            pltpu.VMEM((SEQ * D,), jnp.float32),
            pltpu.VMEM((D,), jnp.float32),
            pltpu.VMEM((NBUF, CHUNK, D), jnp.float32),
        ] + [pltpu.SemaphoreType.DMA] * (2 * NBUF + 1),
    )(tok_emb, tok_idx, seg_x, base_tab, delta)
    return out.reshape(BATCH, SEQ, D)


# speedup vs baseline: 2.9816x; 2.9816x over previous
import jax
import jax.numpy as jnp
from jax import lax
from jax.experimental import pallas as pl
from jax.experimental.pallas import tpu as pltpu, tpu_sc as plsc

VOCAB = 100000
D = 128
SEQ = 200
BATCH = 1024
N = BATCH * SEQ
NC = 2
NS = 16
NW = NC * NS
ROWS_W = N // NW
CHUNK = 128
NCHUNK = ROWS_W // CHUNK
NBUF = 6
IR = ROWS_W // 128
IPC = CHUNK // 128


def _body(tok_emb, add_tab, tok_idx, add_idx, out,
          idx_v, aidx_v, rows_v, atab_s, *sems):
    sid = lax.axis_index("s")
    wid = sid * NC + lax.axis_index("c")
    row0 = wid * ROWS_W

    psem = sems[3 * NBUF]
    pltpu.async_copy(tok_idx.at[wid], idx_v, psem)
    pltpu.async_copy(add_idx.at[wid], aidx_v, psem)
    @pl.when(sid == 0)
    def _():
        pltpu.sync_copy(add_tab, atab_s)
    pltpu.make_async_copy(tok_idx.at[wid], idx_v, psem).wait()
    pltpu.make_async_copy(add_idx.at[wid], aidx_v, psem).wait()
    plsc.subcore_barrier()

    gsems = sems[0:NBUF]
    asems = sems[NBUF:2 * NBUF]
    wsems = sems[2 * NBUF:3 * NBUF]

    def start_tok(g, buf):
        for h in range(IPC):
            pltpu.async_copy(tok_emb.at[idx_v.at[g * IPC + h]],
                             rows_v.at[buf].at[pl.ds(h * 128, 128)],
                             gsems[buf])

    def wait_tok(buf):
        for h in range(IPC):
            pltpu.make_async_copy(
                tok_emb.at[idx_v.at[0]],
                rows_v.at[buf].at[pl.ds(0, 128)], gsems[buf]).wait()

    def start_add(g, buf):
        for h in range(IPC):
            pltpu.async_copy(atab_s.at[aidx_v.at[g * IPC + h]],
                             rows_v.at[buf].at[pl.ds(h * 128, 128)],
                             asems[buf], add=True)

    def wait_add(buf):
        for h in range(IPC):
            pltpu.make_async_copy(
                atab_s.at[aidx_v.at[0]],
                rows_v.at[buf].at[pl.ds(0, 128)], asems[buf]).wait()

    def start_write(g, buf):
        pltpu.async_copy(rows_v.at[buf],
                         out.at[pl.ds(row0 + g * CHUNK, CHUNK)], wsems[buf])

    def wait_write(buf):
        pltpu.make_async_copy(
            rows_v.at[buf], out.at[pl.ds(0, CHUNK)], wsems[buf]).wait()

    for g in range(NCHUNK + 2):
        if g < NCHUNK:
            buf = g % NBUF
            if g >= NBUF:
                wait_write(buf)
            start_tok(g, buf)
        if 0 <= g - 1 < NCHUNK:
            b = (g - 1) % NBUF
            wait_tok(b)
            start_add(g - 1, b)
        if 0 <= g - 2 < NCHUNK:
            b = (g - 2) % NBUF
            wait_add(b)
            start_write(g - 2, b)
    for g in range(NCHUNK - NBUF, NCHUNK):
        wait_write(g % NBUF)


def kernel(x, segment_ids, tok_emb, seg_emb, pos_emb):
    tok_idx = x.astype(jnp.int32).reshape(NW, IR, 128)
    pos = jnp.arange(SEQ, dtype=jnp.int32)
    add_idx = (segment_ids.astype(jnp.int32) * SEQ + pos[None, :]).reshape(NW, IR, 128)
    add_tab = (seg_emb[:, None, :] + pos_emb[None, :SEQ, :]).reshape(2 * SEQ, D)

    mesh = plsc.VectorSubcoreMesh(core_axis_name="c", subcore_axis_name="s")
    out = pl.kernel(
        _body,
        out_type=jax.ShapeDtypeStruct((N, D), jnp.float32),
        mesh=mesh,
        scratch_types=[
            pltpu.VMEM((IR, 128), jnp.int32),
            pltpu.VMEM((IR, 128), jnp.int32),
            pltpu.VMEM((NBUF, CHUNK, D), jnp.float32),
            pltpu.VMEM_SHARED((2 * SEQ, D), jnp.float32),
        ] + [pltpu.SemaphoreType.DMA] * (3 * NBUF + 1),
    )(tok_emb, add_tab, tok_idx, add_idx)
    return out.reshape(BATCH, SEQ, D)
